# baseline (speedup 1.0000x reference)
import jax
import jax.numpy as jnp
from jax.experimental import pallas as pl


def _tent_kernel(diag_ref, samp_ref, out_ref):
    d = diag_ref[0]
    x = d[:, 0:1]
    y = d[:, 1:2]
    s = samp_ref[0]
    out_ref[0] = jnp.maximum(
        0.5 * (y - x) - jnp.abs(s[None, :] - 0.5 * (y + x)), 0.0
    )


def kernel(diagrams, samples):
    n, P, _ = diagrams.shape
    S = samples.shape[0]
    samples2d = samples.reshape(1, S)
    out = pl.pallas_call(
        _tent_kernel,
        grid=(n,),
        in_specs=[
            pl.BlockSpec((1, P, 2), lambda i: (i, 0, 0)),
            pl.BlockSpec((1, S), lambda i: (0, 0)),
        ],
        out_specs=pl.BlockSpec((1, P, S), lambda i: (i, 0, 0)),
        out_shape=jax.ShapeDtypeStruct((n, P, S), diagrams.dtype),
    )(diagrams, samples2d)
    return out
